# baseline (speedup 1.0000x reference)
import functools

import jax
import jax.numpy as jnp
from jax import lax
from jax.experimental import pallas as pl
from jax.experimental.pallas import tpu as pltpu
from jax.experimental.pallas import tpu_sc as plsc

VOCAB = 100000
DIM = 64
BATCH = 4096
NUM_CTX = 5

NC = 2
NS = 16
NW = NC * NS
L = 16

BPW = BATCH // NW
PPW = BPW * NUM_CTX
DCAT = 2 * DIM

_mesh = plsc.VectorSubcoreMesh(core_axis_name="c", subcore_axis_name="s")


@functools.partial(
    pl.kernel,
    mesh=_mesh,
    compiler_params=pltpu.CompilerParams(needs_layout_passes=False),
    out_type=jax.ShapeDtypeStruct((NUM_CTX, BATCH), jnp.float32),
    scratch_types=[
        pltpu.VMEM((BPW,), jnp.int32),
        pltpu.VMEM((PPW,), jnp.int32),
        pltpu.VMEM((BPW, DCAT), jnp.float32),
        pltpu.VMEM((PPW, DCAT), jnp.float32),
        pltpu.VMEM((NUM_CTX, BPW), jnp.float32),
        pltpu.VMEM((80 * 17,), jnp.float32),
        pltpu.SemaphoreType.DMA,
    ],
)
def _w2v_sc(tgt_hbm, ctx_hbm, wcat_hbm, out_hbm,
            tgt_v, ctx_v, we_v, ce_v, out_v, t_v, sem):
    wid = lax.axis_index("s") * NC + lax.axis_index("c")
    base = wid * BPW
    pltpu.sync_copy(tgt_hbm.at[pl.ds(base, BPW)], tgt_v)
    for c in range(NUM_CTX):
        pltpu.sync_copy(ctx_hbm.at[pl.ds(c * BATCH + base, BPW)],
                        ctx_v.at[pl.ds(c * BPW, BPW)])
    copies = [pltpu.async_copy(wcat_hbm.at[tgt_v], we_v, sem)]
    for c in range(NUM_CTX):
        copies.append(
            pltpu.async_copy(wcat_hbm.at[ctx_v.at[pl.ds(c * BPW, BPW)]],
                             ce_v.at[pl.ds(c * BPW, BPW)], sem))
    for cp in copies:
        cp.wait()

    iota0 = lax.iota(jnp.int32, L)
    nq = DIM // L

    def super_group(sg, carry):
        b0 = sg * L
        for bb in range(L):
            b = b0 + bb
            w = [we_v[b, pl.ds(q * L, L)] for q in range(nq)]
            for c in range(NUM_CTX):
                r = c * BPW + b
                part = w[0] * ce_v[r, pl.ds(DIM, L)]
                for q in range(1, nq):
                    part = part + w[q] * ce_v[r, pl.ds(DIM + q * L, L)]
                t_v[pl.ds((bb * NUM_CTX + c) * 17, L)] = part
        for c in range(NUM_CTX):
            cbase = (iota0 * NUM_CTX + c) * 17
            acc = plsc.load_gather(t_v, [cbase])
            for l in range(1, L):
                acc = acc + plsc.load_gather(t_v, [cbase + l])
            out_v[c, pl.ds(b0, L)] = acc
        return carry

    lax.fori_loop(0, BPW // L, super_group, 0)

    out_copies = []
    for c in range(NUM_CTX):
        out_copies.append(pltpu.async_copy(
            out_v.at[pl.ds(c, 1)],
            out_hbm.at[pl.ds(c, 1), pl.ds(base, BPW)], sem))
    for cp in out_copies:
        cp.wait()


def kernel(target, context, W_target, W_context):
    tgt = target.reshape(BATCH)
    ctx = context.T.reshape(NUM_CTX * BATCH)
    wcat = jnp.concatenate([W_target, W_context], axis=1)
    out = _w2v_sc(tgt, ctx, wcat)
    return out.T


# speedup vs baseline: 1.2503x; 1.2503x over previous
import functools

import jax
import jax.numpy as jnp
from jax import lax
from jax.experimental import pallas as pl
from jax.experimental.pallas import tpu as pltpu
from jax.experimental.pallas import tpu_sc as plsc

VOCAB = 100000
DIM = 64
BATCH = 4096
NUM_CTX = 5

NC = 2
NS = 16
NW = NC * NS
L = 16

BPW = BATCH // NW
PPW = BPW * NUM_CTX

_mesh = plsc.VectorSubcoreMesh(core_axis_name="c", subcore_axis_name="s")


@functools.partial(
    pl.kernel,
    mesh=_mesh,
    compiler_params=pltpu.CompilerParams(needs_layout_passes=False),
    out_type=jax.ShapeDtypeStruct((NUM_CTX, BATCH), jnp.float32),
    scratch_types=[
        pltpu.VMEM((BPW,), jnp.int32),
        pltpu.VMEM((PPW,), jnp.int32),
        pltpu.SMEM((BPW,), jnp.int32),
        pltpu.SMEM((PPW,), jnp.int32),
        pltpu.VMEM((BPW, DIM), jnp.float32),
        pltpu.VMEM((PPW, DIM), jnp.float32),
        pltpu.VMEM((NUM_CTX, BPW), jnp.float32),
        pltpu.VMEM((80 * 17,), jnp.float32),
        pltpu.SemaphoreType.DMA,
    ],
)
def _w2v_sc(tgt_hbm, ctx_hbm, wt_hbm, wc_hbm, out_hbm,
            tgt_v, ctx_v, tgt_s, ctx_s, we_v, ce_v, out_v, t_v, sem):
    wid = lax.axis_index("s") * NC + lax.axis_index("c")
    base = wid * BPW
    idx_copies = [pltpu.async_copy(tgt_hbm.at[pl.ds(base, BPW)], tgt_v, sem)]
    for c in range(NUM_CTX):
        idx_copies.append(pltpu.async_copy(
            ctx_hbm.at[pl.ds(c * BATCH + base, BPW)],
            ctx_v.at[pl.ds(c * BPW, BPW)], sem))
    for cp in idx_copies:
        cp.wait()

    iota0 = lax.iota(jnp.int32, L)

    def spill(src_v, dst_s, n):
        def spill_group(g, carry):
            vec = src_v[pl.ds(g * L, L)]
            for j in range(L):
                s = jnp.sum(jnp.where(iota0 == j, vec, 0))
                dst_s[g * L + j] = s
            return carry
        lax.fori_loop(0, n // L, spill_group, 0)

    spill(tgt_v, tgt_s, BPW)
    spill(ctx_v, ctx_s, PPW)

    def issue_rows(b, carry):
        pltpu.async_copy(wt_hbm.at[tgt_s[b]], we_v.at[b], sem)
        for c in range(NUM_CTX):
            pltpu.async_copy(wc_hbm.at[ctx_s[c * BPW + b]],
                             ce_v.at[b * NUM_CTX + c], sem)
        return carry

    lax.fori_loop(0, BPW, issue_rows, 0)

    nq = DIM // L

    def super_group(sg, carry):
        b0 = sg * L
        pltpu.make_async_copy(wc_hbm.at[pl.ds(0, 6 * L)],
                              ce_v.at[pl.ds(0, 6 * L)], sem).wait()
        for bb in range(L):
            b = b0 + bb
            w = [we_v[b, pl.ds(q * L, L)] for q in range(nq)]
            for c in range(NUM_CTX):
                r = b * NUM_CTX + c
                part = w[0] * ce_v[r, pl.ds(0, L)]
                for q in range(1, nq):
                    part = part + w[q] * ce_v[r, pl.ds(q * L, L)]
                t_v[pl.ds((bb * NUM_CTX + c) * 17, L)] = part
        for c in range(NUM_CTX):
            cbase = (iota0 * NUM_CTX + c) * 17
            acc = plsc.load_gather(t_v, [cbase])
            for l in range(1, L):
                acc = acc + plsc.load_gather(t_v, [cbase + l])
            out_v[c, pl.ds(b0, L)] = acc
        return carry

    lax.fori_loop(0, BPW // L, super_group, 0)

    out_copies = []
    for c in range(NUM_CTX):
        out_copies.append(pltpu.async_copy(
            out_v.at[pl.ds(c, 1)],
            out_hbm.at[pl.ds(c, 1), pl.ds(base, BPW)], sem))
    for cp in out_copies:
        cp.wait()


def kernel(target, context, W_target, W_context):
    tgt = target.reshape(BATCH)
    ctx = context.T.reshape(NUM_CTX * BATCH)
    out = _w2v_sc(tgt, ctx, W_target, W_context)
    return out.T
